# baseline (speedup 1.0000x reference)
import jax
import jax.numpy as jnp
from jax import lax
from jax.experimental import pallas as pl
from jax.experimental.pallas import tpu as pltpu
from jax.experimental.pallas import tpu_sc as plsc

_B, _L, _DIM = 1024, 512, 128
_N = _B * _L
_VEC = 16
_NVJ = _DIM // _VEC

_info = plsc.get_sparse_core_info()
_NC = _info.num_cores
_NS = _info.num_subcores
_NW = _NC * _NS
_ROWS_PER_W = _N // _NW
_C = 128
_KB = 16
_NBUF = 3

_MAX_LEN = 512
_COMB_ROWS = 2 * _MAX_LEN
_CB_PER_T = _COMB_ROWS // _NS


def _embed_body(tok_hbm, seg_hbm, pos_hbm, ttab, stab, ptab, out_hbm,
                idxt, idxs, idxp, idxc,
                rt0, rt1, rt2, rc0, rc1, rc2, srows, prows, comb_sp,
                st0, st1, st2, sc0, sc1, sc2, sw0, sw1, sw2):
    sid = lax.axis_index("s")
    wid = sid * _NC + lax.axis_index("c")
    base_w = wid * _ROWS_PER_W
    n_chunks = _ROWS_PER_W // _C
    bufs = ((rt0, rc0, st0, sc0, sw0),
            (rt1, rc1, st1, sc1, sw1),
            (rt2, rc2, st2, sc2, sw2))

    cb0 = pl.multiple_of(sid * _CB_PER_T, _CB_PER_T)
    s_of_tile = cb0 // _MAX_LEN
    p0 = pl.multiple_of(lax.rem(cb0, _MAX_LEN), _CB_PER_T)
    pltpu.sync_copy(stab, srows)
    pltpu.sync_copy(ptab.at[pl.ds(p0, _CB_PER_T)], prows)

    def brow(r, carry):
        for j in range(_NVJ):
            sl = pl.ds(j * _VEC, _VEC)
            prows[r, sl] = prows[r, sl] + srows[s_of_tile, sl]
        return carry

    lax.fori_loop(0, _CB_PER_T, brow, 0)
    pltpu.sync_copy(prows, comb_sp.at[pl.ds(cb0, _CB_PER_T)])
    plsc.subcore_barrier()

    def load_idx_block(k):
        blk = pl.multiple_of((base_w + k * _C) // _C, _KB)
        half = pl.multiple_of(lax.rem(k, 2 * _KB), _KB)
        pltpu.sync_copy(tok_hbm.at[pl.ds(blk, _KB)],
                        idxt.at[pl.ds(half, _KB)])
        pltpu.sync_copy(seg_hbm.at[pl.ds(blk, _KB)], idxs)
        pltpu.sync_copy(pos_hbm.at[pl.ds(blk, _KB)], idxp)

        def crow(r, carry):
            for j in range(_C // _VEC):
                sl = pl.ds(j * _VEC, _VEC)
                idxc[half + r, sl] = idxs[r, sl] * _MAX_LEN + idxp[r, sl]
            return carry

        lax.fori_loop(0, _KB, crow, 0)

    def fire(k, rt, rc, semt, semc):
        kb = lax.rem(k, 2 * _KB)
        pltpu.async_copy(ttab.at[idxt.at[kb]], rt, semt)
        pltpu.async_copy(comb_sp.at[idxc.at[kb]], rc, semc)

    def drain_gathers(rt, rc, semt, semc):
        pltpu.make_async_copy(ttab.at[idxt.at[0]], rt, semt).wait()
        pltpu.make_async_copy(comb_sp.at[idxc.at[0]], rc, semc).wait()

    def drain_write(rt, semw):
        pltpu.make_async_copy(rt, out_hbm.at[pl.ds(0, _C)], semw).wait()

    def add_and_write(k, rt, rc, semw):
        def addrow(r, carry):
            for j in range(_NVJ):
                sl = pl.ds(j * _VEC, _VEC)
                plsc.addupdate(rt.at[r, sl], rc[r, sl])
            return carry

        lax.fori_loop(0, _C, addrow, 0)
        pltpu.async_copy(
            rt,
            out_hbm.at[pl.ds(pl.multiple_of(base_w + k * _C, _C), _C)],
            semw)

    load_idx_block(0)
    fire(0, *bufs[0][:4])
    fire(1, *bufs[1][:4])

    def trio(kk, carry):
        for b in range(_NBUF):
            k = kk * _NBUF + b
            rt, rc, semt, semc, semw = bufs[b]
            b2 = (b + 2) % _NBUF
            nrt, nrc, nsemt, nsemc, nsemw = bufs[b2]

            drain_gathers(rt, rc, semt, semc)

            @pl.when(lax.rem(k + 2, _KB) == 0)
            def _():
                load_idx_block(k + 2)

            @pl.when(k >= 1)
            def _():
                drain_write(nrt, nsemw)

            fire(k + 2, nrt, nrc, nsemt, nsemc)
            add_and_write(k, rt, rc, semw)
        return carry

    lax.fori_loop(0, (n_chunks - 2) // _NBUF, trio, 0)

    for k in (n_chunks - 2, n_chunks - 1):
        rt, rc, semt, semc, semw = bufs[k % _NBUF]
        drain_gathers(rt, rc, semt, semc)
        add_and_write(k, rt, rc, semw)

    for k in (n_chunks - 3, n_chunks - 2, n_chunks - 1):
        rt = bufs[k % _NBUF][0]
        semw = bufs[k % _NBUF][4]
        drain_write(rt, semw)


def kernel(tokens, segment_ids, pos_ids, token_table, segment_table, pos_table):
    tok = jnp.reshape(tokens, (_N // _C, _C)).astype(jnp.int32)
    seg = jnp.reshape(segment_ids, (_N // _C, _C)).astype(jnp.int32)
    pos = jnp.reshape(pos_ids, (_N // _C, _C)).astype(jnp.int32)
    mesh = plsc.VectorSubcoreMesh(core_axis_name="c", subcore_axis_name="s")

    out = pl.kernel(
        _embed_body,
        mesh=mesh,
        out_type=jax.ShapeDtypeStruct((_N, _DIM), jnp.float32),
        scratch_types=[
            pltpu.VMEM((2 * _KB, _C), jnp.int32),
            pltpu.VMEM((_KB, _C), jnp.int32),
            pltpu.VMEM((_KB, _C), jnp.int32),
            pltpu.VMEM((2 * _KB, _C), jnp.int32),
            pltpu.VMEM((_C, _DIM), jnp.float32),
            pltpu.VMEM((_C, _DIM), jnp.float32),
            pltpu.VMEM((_C, _DIM), jnp.float32),
            pltpu.VMEM((_C, _DIM), jnp.float32),
            pltpu.VMEM((_C, _DIM), jnp.float32),
            pltpu.VMEM((_C, _DIM), jnp.float32),
            pltpu.VMEM((2, _DIM), jnp.float32),
            pltpu.VMEM((_CB_PER_T, _DIM), jnp.float32),
            pltpu.VMEM_SHARED((_COMB_ROWS, _DIM), jnp.float32),
            pltpu.SemaphoreType.DMA,
            pltpu.SemaphoreType.DMA,
            pltpu.SemaphoreType.DMA,
            pltpu.SemaphoreType.DMA,
            pltpu.SemaphoreType.DMA,
            pltpu.SemaphoreType.DMA,
            pltpu.SemaphoreType.DMA,
            pltpu.SemaphoreType.DMA,
            pltpu.SemaphoreType.DMA,
        ],
    )(tok, seg, pos, token_table, segment_table, pos_table)
    return jnp.reshape(out, (_B, _L, _DIM))


# speedup vs baseline: 1.0025x; 1.0025x over previous
import jax
import jax.numpy as jnp
from jax import lax
from jax.experimental import pallas as pl
from jax.experimental.pallas import tpu as pltpu
from jax.experimental.pallas import tpu_sc as plsc

_B, _L, _DIM = 1024, 512, 128
_N = _B * _L
_VEC = 16
_NVJ = _DIM // _VEC

_info = plsc.get_sparse_core_info()
_NC = _info.num_cores
_NS = _info.num_subcores
_NW = _NC * _NS
_ROWS_PER_W = _N // _NW
_C = 128
_KB = 16
_BLK = _C * _KB

_MAX_LEN = 512
_COMB_ROWS = 2 * _MAX_LEN
_CB_PER_T = _COMB_ROWS // _NS


def _embed_body(tok_hbm, seg_hbm, pos_hbm, ttab, stab, ptab, out_hbm,
                idxt, idxs, idxp, idxc,
                rt0, rt1, rc0, rc1, srows, prows, comb_sp,
                st0, st1, sc0, sc1, sw0, sw1):
    sid = lax.axis_index("s")
    wid = sid * _NC + lax.axis_index("c")
    base_w = wid * _ROWS_PER_W
    n_chunks = _ROWS_PER_W // _C
    bufs = ((rt0, rc0, st0, sc0, sw0), (rt1, rc1, st1, sc1, sw1))

    cb0 = pl.multiple_of(sid * _CB_PER_T, _CB_PER_T)
    s_of_tile = cb0 // _MAX_LEN
    p0 = pl.multiple_of(lax.rem(cb0, _MAX_LEN), _CB_PER_T)
    pltpu.sync_copy(stab, srows)
    pltpu.sync_copy(ptab.at[pl.ds(p0, _CB_PER_T)], prows)

    def brow(r, carry):
        for j in range(_NVJ):
            sl = pl.ds(j * _VEC, _VEC)
            prows[r, sl] = prows[r, sl] + srows[s_of_tile, sl]
        return carry

    lax.fori_loop(0, _CB_PER_T, brow, 0)
    pltpu.sync_copy(prows, comb_sp.at[pl.ds(cb0, _CB_PER_T)])
    plsc.subcore_barrier()

    def load_idx_block(k):
        blk = pl.multiple_of((base_w + k * _C) // _C, _KB)
        pltpu.sync_copy(tok_hbm.at[pl.ds(blk, _KB)], idxt)
        pltpu.sync_copy(seg_hbm.at[pl.ds(blk, _KB)], idxs)
        pltpu.sync_copy(pos_hbm.at[pl.ds(blk, _KB)], idxp)

        def crow(r, carry):
            for j in range(_C // _VEC):
                sl = pl.ds(j * _VEC, _VEC)
                idxc[r, sl] = idxs[r, sl] * _MAX_LEN + idxp[r, sl]
            return carry

        lax.fori_loop(0, _KB, crow, 0)

    def fire(k, rt, rc, semt, semc):
        kb = lax.rem(k, _KB)
        pltpu.async_copy(ttab.at[idxt.at[kb]], rt, semt)
        pltpu.async_copy(comb_sp.at[idxc.at[kb]], rc, semc)

    def drain_gathers(k, rt, rc, semt, semc):
        pltpu.make_async_copy(ttab.at[idxt.at[0]], rt, semt).wait()
        pltpu.make_async_copy(comb_sp.at[idxc.at[0]], rc, semc).wait()

    def drain_write(rt, semw):
        pltpu.make_async_copy(rt, out_hbm.at[pl.ds(0, _C)], semw).wait()

    load_idx_block(0)
    fire(0, *bufs[0][:4])

    def pair(kk, carry):
        for b in range(2):
            k = kk * 2 + b
            rt, rc, semt, semc, semw = bufs[b]
            nrt, nrc, nsemt, nsemc, nsemw = bufs[1 - b]

            drain_gathers(k, rt, rc, semt, semc)

            if b == 1:
                @pl.when(k < n_chunks - 1)
                def _():
                    @pl.when(lax.rem(k + 1, _KB) == 0)
                    def _():
                        load_idx_block(k + 1)

                    drain_write(nrt, nsemw)
                    fire(k + 1, nrt, nrc, nsemt, nsemc)
            else:
                @pl.when(k >= 1)
                def _():
                    drain_write(nrt, nsemw)

                fire(k + 1, nrt, nrc, nsemt, nsemc)

            def addrow(r, carry2):
                for j in range(_NVJ):
                    sl = pl.ds(j * _VEC, _VEC)
                    plsc.addupdate(rt.at[r, sl], rc[r, sl])
                return carry2

            lax.fori_loop(0, _C, addrow, 0)
            pltpu.async_copy(
                rt,
                out_hbm.at[pl.ds(pl.multiple_of(base_w + k * _C, _C), _C)],
                semw)
        return carry

    lax.fori_loop(0, n_chunks // 2, pair, 0)

    drain_write(rt0, sw0)
    drain_write(rt1, sw1)


def kernel(tokens, segment_ids, pos_ids, token_table, segment_table, pos_table):
    tok = jnp.reshape(tokens, (_N // _C, _C)).astype(jnp.int32)
    seg = jnp.reshape(segment_ids, (_N // _C, _C)).astype(jnp.int32)
    pos = jnp.reshape(pos_ids, (_N // _C, _C)).astype(jnp.int32)
    mesh = plsc.VectorSubcoreMesh(core_axis_name="c", subcore_axis_name="s")

    out = pl.kernel(
        _embed_body,
        mesh=mesh,
        out_type=jax.ShapeDtypeStruct((_N, _DIM), jnp.float32),
        scratch_types=[
            pltpu.VMEM((_KB, _C), jnp.int32),
            pltpu.VMEM((_KB, _C), jnp.int32),
            pltpu.VMEM((_KB, _C), jnp.int32),
            pltpu.VMEM((_KB, _C), jnp.int32),
            pltpu.VMEM((_C, _DIM), jnp.float32),
            pltpu.VMEM((_C, _DIM), jnp.float32),
            pltpu.VMEM((_C, _DIM), jnp.float32),
            pltpu.VMEM((_C, _DIM), jnp.float32),
            pltpu.VMEM((2, _DIM), jnp.float32),
            pltpu.VMEM((_CB_PER_T, _DIM), jnp.float32),
            pltpu.VMEM_SHARED((_COMB_ROWS, _DIM), jnp.float32),
            pltpu.SemaphoreType.DMA,
            pltpu.SemaphoreType.DMA,
            pltpu.SemaphoreType.DMA,
            pltpu.SemaphoreType.DMA,
            pltpu.SemaphoreType.DMA,
            pltpu.SemaphoreType.DMA,
        ],
    )(tok, seg, pos, token_table, segment_table, pos_table)
    return jnp.reshape(out, (_B, _L, _DIM))
